# Initial kernel scaffold; baseline (speedup 1.0000x reference)
import jax
import jax.numpy as jnp
from jax.experimental import pallas as pl


def kernel(indices, amplitude_table, phase_table):
    raise NotImplementedError("write your pallas kernel here")


# speedup vs baseline: 1.0951x; 1.0951x over previous
import functools

import jax
import jax.numpy as jnp
from jax import lax
from jax.experimental import pallas as pl
from jax.experimental.pallas import tpu as pltpu
from jax.experimental.pallas import tpu_sc as plsc

_NUM_CORES = 2
_NUM_SUBCORES = 16
_NUM_WORKERS = _NUM_CORES * _NUM_SUBCORES


@functools.partial(jax.jit, static_argnums=())
def kernel(indices, amplitude_table, phase_table):
    B, H = indices.shape
    V, D = amplitude_table.shape
    N = B * H

    assert N % _NUM_WORKERS == 0
    rows_per_worker = N // _NUM_WORKERS

    chunk = 640
    while rows_per_worker % chunk:
        chunk //= 2
    steps = rows_per_worker // chunk

    flat_idx = indices.reshape(N).astype(jnp.int32)

    mesh = plsc.VectorSubcoreMesh(core_axis_name="c", subcore_axis_name="s")
    out_sds = jax.ShapeDtypeStruct((N, D), jnp.float32)

    @functools.partial(
        pl.kernel,
        mesh=mesh,
        out_type=[out_sds, out_sds],
        scratch_types=[
            pltpu.VMEM((chunk,), jnp.int32),
            pltpu.VMEM((chunk, D), jnp.float32),
            pltpu.VMEM((chunk, D), jnp.float32),
            pltpu.SemaphoreType.DMA,
            pltpu.SemaphoreType.DMA,
        ],
        compiler_params=pltpu.CompilerParams(use_tc_tiling_on_sc=False),
    )
    def sc_kernel(idx_hbm, amp_hbm, ph_hbm, amp_out, ph_out,
                  idx_v, amp_v, ph_v, sem_a, sem_p):
        wid = lax.axis_index("s") * _NUM_CORES + lax.axis_index("c")
        base = wid * rows_per_worker

        @pl.loop(0, steps)
        def _(i):
            off = base + i * chunk
            pltpu.sync_copy(idx_hbm.at[pl.ds(off, chunk)], idx_v)
            a = pltpu.async_copy(amp_hbm.at[idx_v], amp_v, sem_a)
            p = pltpu.async_copy(ph_hbm.at[idx_v], ph_v, sem_p)
            a.wait()
            p.wait()
            pltpu.sync_copy(amp_v, amp_out.at[pl.ds(off, chunk)])
            pltpu.sync_copy(ph_v, ph_out.at[pl.ds(off, chunk)])

    amp_flat, ph_flat = sc_kernel(flat_idx, amplitude_table, phase_table)
    return (amp_flat.reshape(B, H, D), ph_flat.reshape(B, H, D))


# speedup vs baseline: 1.0995x; 1.0041x over previous
import functools

import jax
import jax.numpy as jnp
from jax import lax
from jax.experimental import pallas as pl
from jax.experimental.pallas import tpu as pltpu
from jax.experimental.pallas import tpu_sc as plsc

_NUM_CORES = 2
_NUM_SUBCORES = 16
_NUM_WORKERS = _NUM_CORES * _NUM_SUBCORES
_NBUF = 2


@jax.jit
def kernel(indices, amplitude_table, phase_table):
    B, H = indices.shape
    V, D = amplitude_table.shape
    N = B * H

    assert N % _NUM_WORKERS == 0
    rows_per_worker = N // _NUM_WORKERS

    chunk = 400
    while rows_per_worker % chunk:
        chunk //= 2
    steps = rows_per_worker // chunk

    flat_idx = indices.reshape(N).astype(jnp.int32)

    mesh = plsc.VectorSubcoreMesh(core_axis_name="c", subcore_axis_name="s")
    out_sds = jax.ShapeDtypeStruct((N, D), jnp.float32)

    scratch = (
        [pltpu.VMEM((chunk,), jnp.int32) for _ in range(_NBUF)]
        + [pltpu.VMEM((chunk, D), jnp.float32) for _ in range(2 * _NBUF)]
        + [pltpu.SemaphoreType.DMA for _ in range(4 * _NBUF)]
    )

    @functools.partial(
        pl.kernel,
        mesh=mesh,
        out_type=[out_sds, out_sds],
        scratch_types=scratch,
        compiler_params=pltpu.CompilerParams(use_tc_tiling_on_sc=False),
    )
    def sc_kernel(idx_hbm, amp_hbm, ph_hbm, amp_out, ph_out, *s):
        idx_v = s[0:_NBUF]
        amp_v = s[_NBUF:2 * _NBUF]
        ph_v = s[2 * _NBUF:3 * _NBUF]
        sg_a = s[3 * _NBUF:4 * _NBUF]
        sg_p = s[4 * _NBUF:5 * _NBUF]
        sw_a = s[5 * _NBUF:6 * _NBUF]
        sw_p = s[6 * _NBUF:7 * _NBUF]

        wid = lax.axis_index("s") * _NUM_CORES + lax.axis_index("c")
        base = wid * rows_per_worker

        def load_and_gather(i, b):
            off = base + i * chunk
            pltpu.sync_copy(idx_hbm.at[pl.ds(off, chunk)], idx_v[b])
            ha = pltpu.async_copy(amp_hbm.at[idx_v[b]], amp_v[b], sg_a[b])
            hp = pltpu.async_copy(ph_hbm.at[idx_v[b]], ph_v[b], sg_p[b])
            return ha, hp

        gathers = [None] * _NBUF
        writes = [None] * _NBUF
        gathers[0] = load_and_gather(0, 0)
        for i in range(steps):
            b = i % _NBUF
            if i + 1 < steps:
                nb = (i + 1) % _NBUF
                if writes[nb] is not None:
                    writes[nb][0].wait()
                    writes[nb][1].wait()
                    writes[nb] = None
                gathers[nb] = load_and_gather(i + 1, nb)
            gathers[b][0].wait()
            gathers[b][1].wait()
            off = base + i * chunk
            wa = pltpu.async_copy(amp_v[b], amp_out.at[pl.ds(off, chunk)],
                                  sw_a[b])
            wp = pltpu.async_copy(ph_v[b], ph_out.at[pl.ds(off, chunk)],
                                  sw_p[b])
            writes[b] = (wa, wp)
        for w in writes:
            if w is not None:
                w[0].wait()
                w[1].wait()

    amp_flat, ph_flat = sc_kernel(flat_idx, amplitude_table, phase_table)
    return (amp_flat.reshape(B, H, D), ph_flat.reshape(B, H, D))


# speedup vs baseline: 1.0995x; 1.0000x over previous
import functools

import jax
import jax.numpy as jnp
from jax import lax
from jax.experimental import pallas as pl
from jax.experimental.pallas import tpu as pltpu
from jax.experimental.pallas import tpu_sc as plsc

_NUM_CORES = 2
_NUM_SUBCORES = 16
_NUM_WORKERS = _NUM_CORES * _NUM_SUBCORES
_NBUF = 2
_BCHUNK = 8


@jax.jit
def kernel(indices, amplitude_table, phase_table):
    B, H = indices.shape
    V, D = amplitude_table.shape

    assert B % (_NUM_WORKERS * _BCHUNK) == 0
    batches_per_worker = B // _NUM_WORKERS
    steps = batches_per_worker // _BCHUNK
    chunk = _BCHUNK * H

    idx32 = indices.reshape(B * H).astype(jnp.int32)

    mesh = plsc.VectorSubcoreMesh(core_axis_name="c", subcore_axis_name="s")
    out_sds = jax.ShapeDtypeStruct((B, H, D), jnp.float32)

    scratch = (
        [pltpu.VMEM((chunk,), jnp.int32) for _ in range(_NBUF)]
        + [pltpu.VMEM((chunk, D), jnp.float32) for _ in range(2 * _NBUF)]
        + [pltpu.SemaphoreType.DMA for _ in range(4 * _NBUF)]
    )

    @functools.partial(
        pl.kernel,
        mesh=mesh,
        out_type=[out_sds, out_sds],
        scratch_types=scratch,
        compiler_params=pltpu.CompilerParams(use_tc_tiling_on_sc=False),
    )
    def sc_kernel(idx_hbm, amp_hbm, ph_hbm, amp_out, ph_out, *s):
        idx_v = s[0:_NBUF]
        amp_v = s[_NBUF:2 * _NBUF]
        ph_v = s[2 * _NBUF:3 * _NBUF]
        sg_a = s[3 * _NBUF:4 * _NBUF]
        sg_p = s[4 * _NBUF:5 * _NBUF]
        sw_a = s[5 * _NBUF:6 * _NBUF]
        sw_p = s[6 * _NBUF:7 * _NBUF]

        wid = lax.axis_index("s") * _NUM_CORES + lax.axis_index("c")
        batch_base = wid * batches_per_worker
        row_base = batch_base * H

        def load_and_gather(i, b):
            off = row_base + i * chunk
            pltpu.sync_copy(idx_hbm.at[pl.ds(off, chunk)], idx_v[b])
            ha = pltpu.async_copy(amp_hbm.at[idx_v[b]], amp_v[b], sg_a[b])
            hp = pltpu.async_copy(ph_hbm.at[idx_v[b]], ph_v[b], sg_p[b])
            return ha, hp

        gathers = [None] * _NBUF
        writes = [None] * _NBUF
        gathers[0] = load_and_gather(0, 0)
        for i in range(steps):
            b = i % _NBUF
            if i + 1 < steps:
                nb = (i + 1) % _NBUF
                if writes[nb] is not None:
                    for w in writes[nb]:
                        w.wait()
                    writes[nb] = None
                gathers[nb] = load_and_gather(i + 1, nb)
            gathers[b][0].wait()
            gathers[b][1].wait()
            b0 = batch_base + i * _BCHUNK
            ws = []
            for j in range(_BCHUNK):
                ws.append(pltpu.async_copy(
                    amp_v[b].at[pl.ds(j * H, H), :], amp_out.at[b0 + j],
                    sw_a[b]))
                ws.append(pltpu.async_copy(
                    ph_v[b].at[pl.ds(j * H, H), :], ph_out.at[b0 + j],
                    sw_p[b]))
            writes[b] = ws
        for ws_ in writes:
            if ws_ is not None:
                for w in ws_:
                    w.wait()

    amp_e, ph_e = sc_kernel(idx32, amplitude_table, phase_table)
    return (amp_e, ph_e)


# speedup vs baseline: 1.1144x; 1.0135x over previous
import functools

import jax
import jax.numpy as jnp
from jax import lax
from jax.experimental import pallas as pl
from jax.experimental.pallas import tpu as pltpu
from jax.experimental.pallas import tpu_sc as plsc

_NUM_CORES = 2
_NUM_SUBCORES = 16
_NUM_WORKERS = _NUM_CORES * _NUM_SUBCORES
_NBUF = 2


@jax.jit
def kernel(indices, amplitude_table, phase_table):
    B, H = indices.shape
    V, D = amplitude_table.shape
    N = B * H

    assert N % _NUM_WORKERS == 0
    rows_per_worker = N // _NUM_WORKERS

    chunk = 400
    while rows_per_worker % chunk:
        chunk //= 2
    steps = rows_per_worker // chunk

    idx32 = indices.reshape(N).astype(jnp.int32)
    packed_table = jnp.concatenate([amplitude_table, phase_table], axis=1)

    mesh = plsc.VectorSubcoreMesh(core_axis_name="c", subcore_axis_name="s")
    out_sds = jax.ShapeDtypeStruct((N, 2 * D), jnp.float32)

    scratch = (
        [pltpu.VMEM((chunk,), jnp.int32) for _ in range(_NBUF)]
        + [pltpu.VMEM((chunk, 2 * D), jnp.float32) for _ in range(_NBUF)]
        + [pltpu.SemaphoreType.DMA for _ in range(2 * _NBUF)]
    )

    @functools.partial(
        pl.kernel,
        mesh=mesh,
        out_type=out_sds,
        scratch_types=scratch,
        compiler_params=pltpu.CompilerParams(use_tc_tiling_on_sc=False),
    )
    def sc_kernel(idx_hbm, tbl_hbm, out_hbm, *s):
        idx_v = s[0:_NBUF]
        row_v = s[_NBUF:2 * _NBUF]
        sg = s[2 * _NBUF:3 * _NBUF]
        sw = s[3 * _NBUF:4 * _NBUF]

        wid = lax.axis_index("s") * _NUM_CORES + lax.axis_index("c")
        base = wid * rows_per_worker

        def load_and_gather(i, b):
            off = base + i * chunk
            pltpu.sync_copy(idx_hbm.at[pl.ds(off, chunk)], idx_v[b])
            return pltpu.async_copy(tbl_hbm.at[idx_v[b]], row_v[b], sg[b])

        gathers = [None] * _NBUF
        writes = [None] * _NBUF
        gathers[0] = load_and_gather(0, 0)
        for i in range(steps):
            b = i % _NBUF
            if i + 1 < steps:
                nb = (i + 1) % _NBUF
                if writes[nb] is not None:
                    writes[nb].wait()
                    writes[nb] = None
                gathers[nb] = load_and_gather(i + 1, nb)
            gathers[b].wait()
            off = base + i * chunk
            writes[b] = pltpu.async_copy(
                row_v[b], out_hbm.at[pl.ds(off, chunk)], sw[b])
        for w in writes:
            if w is not None:
                w.wait()

    packed = sc_kernel(idx32, packed_table)
    amp_e = packed[:, :D].reshape(B, H, D)
    ph_e = packed[:, D:].reshape(B, H, D)
    return (amp_e, ph_e)
